# Initial kernel scaffold; baseline (speedup 1.0000x reference)
import jax
import jax.numpy as jnp
from jax.experimental import pallas as pl


def kernel(x, y, x_pos, y_pos):
    raise NotImplementedError("write your pallas kernel here")


# speedup vs baseline: 3840.5991x; 3840.5991x over previous
import functools

import jax
import jax.numpy as jnp
from jax.experimental import pallas as pl

B, N, M = 4096, 2048, 2048
LOGW = 12
W = N + M
R = 64


def _partner(arr, d, bit_set):
    fwd = jnp.roll(arr, -d, axis=-1)
    bwd = jnp.roll(arr, d, axis=-1)
    return jnp.where(bit_set, bwd, fwd)


def _wass_kernel(x_ref, y_ref, xp_ref, yp_ref, out_ref):
    x = x_ref[...]
    y = y_ref[...]
    key = jnp.concatenate([xp_ref[...], yp_ref[...]], axis=1)
    sx = jnp.sum(x, axis=1, keepdims=True)
    sy = jnp.sum(y, axis=1, keepdims=True)
    val = jnp.concatenate([x / sx, -(y / sy)], axis=1)

    lane = jax.lax.broadcasted_iota(jnp.int32, (R, W), 1)

    for k in range(1, LOGW + 1):
        up = ((lane >> k) & 1) == 0
        for j in reversed(range(k)):
            d = 1 << j
            bit = ((lane >> j) & 1) == 1
            take_min = jnp.logical_xor(bit, up)
            pk = _partner(key, d, bit)
            pv = _partner(val, d, bit)
            nk = jnp.where(take_min, jnp.minimum(key, pk), jnp.maximum(key, pk))
            changed = nk != key
            val = jnp.where(changed, pv, val)
            key = nk

    csum = val
    for j in range(LOGW):
        s = 1 << j
        csum = csum + jnp.where(lane >= s, jnp.roll(csum, s, axis=-1), 0.0)

    nxt = jnp.roll(key, -1, axis=-1)
    delta = jnp.where(lane < W - 1, nxt - key, 0.0)
    loss = jnp.sum(delta * jnp.abs(csum), axis=1)
    out_ref[...] = loss[None, None, :]


@jax.jit
def kernel(x, y, x_pos, y_pos):
    b = x.shape[0]
    grid = (b // R,)
    out = pl.pallas_call(
        _wass_kernel,
        grid=grid,
        in_specs=[
            pl.BlockSpec((R, N), lambda i: (i, 0)),
            pl.BlockSpec((R, M), lambda i: (i, 0)),
            pl.BlockSpec((R, N), lambda i: (i, 0)),
            pl.BlockSpec((R, M), lambda i: (i, 0)),
        ],
        out_specs=pl.BlockSpec((1, 1, R), lambda i: (i, 0, 0)),
        out_shape=jax.ShapeDtypeStruct((b // R, 1, R), jnp.float32),
    )(x, y, x_pos, y_pos)
    return out.reshape(b)
